# baseline (speedup 1.0000x reference)
import functools

import jax
import jax.numpy as jnp
from jax import lax
from jax.experimental import pallas as pl
from jax.experimental.pallas import tpu as pltpu
from jax.experimental.pallas import tpu_sc as plsc

NC, NS, L = 2, 16, 16
NW = NC * NS
B, T, D = 1024, 50, 128
BT = B * T
PER_W = BT // NW
CH = 200
NCHUNK = PER_W // CH
NBUF = 4


def _sc_gather_body(ids_hbm, table_hbm, out_hbm, idx_v,
                    b0, b1, b2, b3, g0, g1, g2, g3, s0, s1, s2, s3):
    wid = lax.axis_index("s") * NC + lax.axis_index("c")
    base = wid * PER_W
    pltpu.sync_copy(ids_hbm.at[pl.ds(base, PER_W)], idx_v)
    bufs = (b0, b1, b2, b3)
    gsem = (g0, g1, g2, g3)
    ssem = (s0, s1, s2, s3)

    def start_gather(c):
        return pltpu.async_copy(
            table_hbm.at[idx_v.at[pl.ds(c * CH, CH)]],
            bufs[c % NBUF], gsem[c % NBUF])

    AHEAD = 2
    gh = [None] * NCHUNK
    sh = [None] * NCHUNK
    for c in range(AHEAD):
        gh[c] = start_gather(c)
    for c in range(NCHUNK):
        n = c + AHEAD
        if n < NCHUNK:
            p = n - NBUF
            if p >= 0:
                sh[p].wait()
            gh[n] = start_gather(n)
        gh[c].wait()
        sh[c] = pltpu.async_copy(
            bufs[c % NBUF], out_hbm.at[pl.ds(base + c * CH, CH)],
            ssem[c % NBUF])
    for c in range(NCHUNK - NBUF, NCHUNK):
        sh[c].wait()


@functools.cache
def _sc_gather():
    return pl.kernel(
        _sc_gather_body,
        out_type=jax.ShapeDtypeStruct((BT, D), jnp.float32),
        mesh=plsc.VectorSubcoreMesh(
            core_axis_name="c", subcore_axis_name="s", num_cores=NC),
        scratch_types=[
            pltpu.VMEM((PER_W,), jnp.int32),
        ] + [pltpu.VMEM((CH, D), jnp.float32)] * NBUF
          + [pltpu.SemaphoreType.DMA] * (2 * NBUF),
    )


_SQRT_HALF = 0.7071067811865476


def _gelu(x):
    return 0.5 * x * (1.0 + lax.erf(x * _SQRT_HALF))


def _dotT(a, b):
    return lax.dot_general(a, b, (((0,), (0,)), ((), ())),
                           preferred_element_type=jnp.float32)


def _tc_body(tok_ref, cat_ref, num_ref, time_ref, base_ref,
             ttab_ref, cstab_ref,
             ctab_hbm, pw_hbm,
             nw1_ref, nb1_ref, nw2_ref, nb2_ref, ng_ref, ngb_ref,
             tw1_ref, tb1_ref, tw2_ref, tb2_ref, tg_ref, tgb_ref,
             pb_ref, lng_ref, lnb_ref,
             out_ref,
             counts_ref, ctab_s, pw_s, sem0, sem1):
    g = pl.program_id(0)

    @pl.when(g == 0)
    def _init():
        counts_ref[...] = jnp.zeros((1, B), jnp.float32)
        c1 = pltpu.make_async_copy(ctab_hbm, ctab_s, sem0)
        c2 = pltpu.make_async_copy(pw_hbm, pw_s, sem1)
        c1.start()
        c2.start()
        c1.wait()
        c2.wait()

    tok = tok_ref[0]
    base = base_ref[0]

    cnt = counts_ref[...] + (tok == 6).astype(jnp.float32)
    counts_ref[...] = cnt

    out_ref[0] = base + (ttab_ref[6:7, :] + cstab_ref[0:1, :])

    @pl.when(jnp.any(cnt > 0))
    def _case_active():
        case_id = jnp.minimum(cnt.astype(jnp.int32), 31)
        oh_c = (case_id == lax.broadcasted_iota(jnp.int32, (32, B), 0))
        casec = _dotT(oh_c.astype(jnp.float32), cstab_ref[...])
        out_ref[0] = out_ref[0] + (casec - cstab_ref[0:1, :])

    @pl.when(jnp.any(tok < 7))
    def _special_types():
        tid = jnp.where(tok == 1, 1, jnp.zeros_like(tok))
        tid = jnp.where(tok == 2, 2, tid)
        tid = jnp.where(tok == 3, 3, tid)
        tid = jnp.where((tok == 4) | (tok == 5), 4, tid)
        tid = jnp.where(tok == 6, 5, tid)
        tid = jnp.where(tok >= 7, 6, tid)
        oh_t = (tid == lax.broadcasted_iota(jnp.int32, (8, B), 0))
        typec = _dotT(oh_t.astype(jnp.float32), ttab_ref[...])
        out_ref[0] = out_ref[0] + (typec - ttab_ref[6:7, :])

    @pl.when(jnp.any(tok == 1))
    def _event():
        def mlp(x, gm, gb, w1, b1, w2, b2):
            m = jnp.mean(x, axis=0, keepdims=True)
            v = jnp.mean((x - m) ** 2, axis=0, keepdims=True)
            xn = (x - m) / jnp.sqrt(v + 1e-5) * gm + gb
            h = _gelu(_dotT(xn, w1) + b1)
            return jnp.dot(h, w2,
                           preferred_element_type=jnp.float32) + b2

        num_h = mlp(num_ref[0], ng_ref[...], ngb_ref[...],
                    nw1_ref[...], nb1_ref[...], nw2_ref[...], nb2_ref[...])
        time_h = mlp(time_ref[0], tg_ref[...], tgb_ref[...],
                     tw1_ref[...], tb1_ref[...], tw2_ref[...], tb2_ref[...])

        ev = (pb_ref[...]
              + jnp.dot(num_h, pw_s[256:320, :],
                        preferred_element_type=jnp.float32)
              + jnp.dot(time_h, pw_s[320:384, :],
                        preferred_element_type=jnp.float32))

        cat = cat_ref[0]
        sub = lax.broadcasted_iota(jnp.int32, (128, B), 0)
        for t in range(4):
            row = cat[t:t + 1, :]
            ck = jnp.zeros((B, 64), jnp.float32)
            for c in range(8):
                oh = (row == sub + c * 128).astype(jnp.float32)
                ck = ck + _dotT(oh, ctab_s[t, c * 128:(c + 1) * 128, :])
            ev = ev + jnp.dot(ck, pw_s[64 * t:64 * (t + 1), :],
                              preferred_element_type=jnp.float32)

        ev = _gelu(ev)
        m = jnp.mean(ev, axis=1, keepdims=True)
        v = jnp.mean((ev - m) ** 2, axis=1, keepdims=True)
        ev = (ev - m) / jnp.sqrt(v + 1e-5) * lng_ref[...] + lnb_ref[...]

        is_row = (tok == 1).astype(jnp.float32)
        is_col = _dotT(is_row, jnp.ones((1, 1), jnp.float32))
        out_ref[0] = out_ref[0] + is_col * ev


def _t_spec(shape):
    nd = len(shape)
    return pl.BlockSpec((1,) + shape[1:],
                        lambda g: (g,) + (0,) * (nd - 1))


def _w_spec(shape):
    nd = len(shape)
    return pl.BlockSpec(shape, lambda g, _n=nd: (0,) * _n)


_ANY = pl.BlockSpec(memory_space=pltpu.MemorySpace.HBM)


def kernel(token_ids, cat_feats, num_feats, time_feats, tok_table,
           cat_tables, num_norm_g, num_norm_b, time_norm_g, time_norm_b,
           num_w1, num_b1, num_w2, num_b2, time_w1, time_b1, time_w2,
           time_b2, proj_w, proj_b, proj_ln_g, proj_ln_b, type_table,
           case_table, event_scale, type_scale, case_scale):
    ids_flat = jnp.transpose(token_ids, (1, 0)).reshape(BT).astype(jnp.int32)
    base = _sc_gather()(ids_flat, tok_table)

    ttab = jnp.pad(type_table * type_scale, ((0, 1), (0, 0)))
    cstab = case_table * case_scale
    lng = proj_ln_g * event_scale
    lnb = proj_ln_b * event_scale
    ctab = jnp.pad(cat_tables, ((0, 0), (0, 24), (0, 0)))

    r2 = lambda a: a.reshape(1, -1)
    c2 = lambda a: a.reshape(-1, 1)
    out = pl.pallas_call(
        _tc_body,
        grid=(T,),
        in_specs=[
            _t_spec((T, 1, B)), _t_spec((T, 4, B)), _t_spec((T, 16, B)),
            _t_spec((T, 8, B)), _t_spec((T, B, D)),
            _w_spec((8, D)), _w_spec((32, D)),
            _ANY, _ANY,
            _w_spec((16, 64)), _w_spec((1, 64)),
            _w_spec((64, 64)), _w_spec((1, 64)),
            _w_spec((16, 1)), _w_spec((16, 1)),
            _w_spec((8, 64)), _w_spec((1, 64)),
            _w_spec((64, 64)), _w_spec((1, 64)),
            _w_spec((8, 1)), _w_spec((8, 1)),
            _w_spec((1, D)), _w_spec((1, D)), _w_spec((1, D)),
        ],
        out_specs=_t_spec((T, B, D)),
        out_shape=jax.ShapeDtypeStruct((T, B, D), jnp.float32),
        scratch_shapes=[
            pltpu.VMEM((1, B), jnp.float32),
            pltpu.VMEM((4, 1024, 64), jnp.float32),
            pltpu.VMEM((384, D), jnp.float32),
            pltpu.SemaphoreType.DMA,
            pltpu.SemaphoreType.DMA,
        ],
        compiler_params=pltpu.CompilerParams(
            dimension_semantics=("arbitrary",)),
    )(
        jnp.transpose(token_ids, (1, 0)).reshape(T, 1, B).astype(jnp.int32),
        jnp.transpose(cat_feats, (1, 2, 0)).astype(jnp.int32),
        jnp.transpose(num_feats, (1, 2, 0)),
        jnp.transpose(time_feats, (1, 2, 0)),
        base.reshape(T, B, D),
        ttab, cstab,
        ctab, proj_w,
        num_w1, r2(num_b1), num_w2, r2(num_b2),
        c2(num_norm_g), c2(num_norm_b),
        time_w1, r2(time_b1), time_w2, r2(time_b2),
        c2(time_norm_g), c2(time_norm_b),
        r2(proj_b), r2(lng), r2(lnb),
    )
    return jnp.transpose(out, (1, 0, 2))


# speedup vs baseline: 1.2196x; 1.2196x over previous
import functools

import jax
import jax.numpy as jnp
from jax import lax
from jax.experimental import pallas as pl
from jax.experimental.pallas import tpu as pltpu
from jax.experimental.pallas import tpu_sc as plsc

NC, NS, L = 2, 16, 16
NW = NC * NS
B, T, D = 1024, 50, 128
BT = B * T
PER_W = BT // NW
CH = 200
NCHUNK = PER_W // CH
NBUF = 4

TB = 2
GRID = T // TB


def _sc_gather_body(ids_hbm, table_hbm, out_hbm, idx_v,
                    b0, b1, b2, b3, g0, g1, g2, g3, s0, s1, s2, s3):
    wid = lax.axis_index("s") * NC + lax.axis_index("c")
    base = wid * PER_W
    pltpu.sync_copy(ids_hbm.at[pl.ds(base, PER_W)], idx_v)
    bufs = (b0, b1, b2, b3)
    gsem = (g0, g1, g2, g3)
    ssem = (s0, s1, s2, s3)

    def start_gather(c):
        return pltpu.async_copy(
            table_hbm.at[idx_v.at[pl.ds(c * CH, CH)]],
            bufs[c % NBUF], gsem[c % NBUF])

    AHEAD = 2
    gh = [None] * NCHUNK
    sh = [None] * NCHUNK
    for c in range(AHEAD):
        gh[c] = start_gather(c)
    for c in range(NCHUNK):
        n = c + AHEAD
        if n < NCHUNK:
            p = n - NBUF
            if p >= 0:
                sh[p].wait()
            gh[n] = start_gather(n)
        gh[c].wait()
        sh[c] = pltpu.async_copy(
            bufs[c % NBUF], out_hbm.at[pl.ds(base + c * CH, CH)],
            ssem[c % NBUF])
    for c in range(NCHUNK - NBUF, NCHUNK):
        sh[c].wait()


@functools.cache
def _sc_gather():
    return pl.kernel(
        _sc_gather_body,
        out_type=jax.ShapeDtypeStruct((BT, D), jnp.float32),
        mesh=plsc.VectorSubcoreMesh(
            core_axis_name="c", subcore_axis_name="s", num_cores=NC),
        scratch_types=[
            pltpu.VMEM((PER_W,), jnp.int32),
        ] + [pltpu.VMEM((CH, D), jnp.float32)] * NBUF
          + [pltpu.SemaphoreType.DMA] * (2 * NBUF),
    )


_SQRT_HALF = 0.7071067811865476


def _gelu(x):
    return 0.5 * x * (1.0 + lax.erf(x * _SQRT_HALF))


def _dotT(a, b):
    return lax.dot_general(a, b, (((0,), (0,)), ((), ())),
                           preferred_element_type=jnp.float32)


def _tc_body(tok_ref, base_ref,
             ttab_ref, cstab_ref,
             cat_hbm, num_hbm, time_hbm, ctab_hbm, pw_hbm,
             nw1_ref, nb1_ref, nw2_ref, nb2_ref, ng_ref, ngb_ref,
             tw1_ref, tb1_ref, tw2_ref, tb2_ref, tg_ref, tgb_ref,
             pb_ref, lng_ref, lnb_ref,
             out_ref,
             counts_ref, ctab_s, pw_s, cat_s, num_s, time_s,
             sem0, sem1, sem2):
    g = pl.program_id(0)

    @pl.when(g == 0)
    def _init():
        counts_ref[...] = jnp.zeros((1, B), jnp.float32)
        c1 = pltpu.make_async_copy(ctab_hbm, ctab_s, sem0)
        c2 = pltpu.make_async_copy(pw_hbm, pw_s, sem1)
        c1.start()
        c2.start()
        c1.wait()
        c2.wait()

    tok = tok_ref[:, 0, :]

    m6 = (tok == 6).astype(jnp.float32)
    cnts = []
    c = counts_ref[...]
    for i in range(TB):
        c = c + m6[i:i + 1, :]
        cnts.append(c)
    counts_ref[...] = c

    out_ref[...] = base_ref[...] + (ttab_ref[6:7, :]
                                    + cstab_ref[0:1, :])[None]

    case_on = jnp.max(c) > 0.0
    r = jnp.max(jnp.where(tok < 7,
                          jnp.where(tok == 1, 6, 2),
                          jnp.zeros_like(tok)))

    @pl.when(case_on)
    def _case_active():
        for i in range(TB):
            case_id = jnp.minimum(cnts[i].astype(jnp.int32), 31)
            oh_c = (case_id
                    == lax.broadcasted_iota(jnp.int32, (32, B), 0))
            casec = _dotT(oh_c.astype(jnp.float32), cstab_ref[...])
            out_ref[i] = out_ref[i] + (casec - cstab_ref[0:1, :])

    @pl.when((r & 2) != 0)
    def _special_types():
        for i in range(TB):
            ti = tok[i:i + 1, :]
            tid = jnp.where(ti == 1, 1, jnp.zeros_like(ti))
            tid = jnp.where(ti == 2, 2, tid)
            tid = jnp.where(ti == 3, 3, tid)
            tid = jnp.where((ti == 4) | (ti == 5), 4, tid)
            tid = jnp.where(ti == 6, 5, tid)
            tid = jnp.where(ti >= 7, 6, tid)
            oh_t = (tid == lax.broadcasted_iota(jnp.int32, (8, B), 0))
            typec = _dotT(oh_t.astype(jnp.float32), ttab_ref[...])
            out_ref[i] = out_ref[i] + (typec - ttab_ref[6:7, :])

    @pl.when((r & 4) != 0)
    def _event():
        f1 = pltpu.make_async_copy(cat_hbm.at[pl.ds(g * TB, TB)],
                                   cat_s, sem0)
        f2 = pltpu.make_async_copy(num_hbm.at[pl.ds(g * TB, TB)],
                                   num_s, sem1)
        f3 = pltpu.make_async_copy(time_hbm.at[pl.ds(g * TB, TB)],
                                   time_s, sem2)
        f1.start()
        f2.start()
        f3.start()
        f1.wait()
        f2.wait()
        f3.wait()

        def mlp(x, gm, gb, w1, b1, w2, b2):
            m = jnp.mean(x, axis=0, keepdims=True)
            va = jnp.mean((x - m) ** 2, axis=0, keepdims=True)
            xn = (x - m) / jnp.sqrt(va + 1e-5) * gm + gb
            h = _gelu(_dotT(xn, w1) + b1)
            return jnp.dot(h, w2,
                           preferred_element_type=jnp.float32) + b2

        for i in range(TB):
            ti = tok[i:i + 1, :]

            @pl.when(jnp.any(ti == 1))
            def _event_row(i=i, ti=ti):
                num_h = mlp(num_s[i], ng_ref[...], ngb_ref[...],
                            nw1_ref[...], nb1_ref[...],
                            nw2_ref[...], nb2_ref[...])
                time_h = mlp(time_s[i], tg_ref[...], tgb_ref[...],
                             tw1_ref[...], tb1_ref[...],
                             tw2_ref[...], tb2_ref[...])

                ev = (pb_ref[...]
                      + jnp.dot(num_h, pw_s[256:320, :],
                                preferred_element_type=jnp.float32)
                      + jnp.dot(time_h, pw_s[320:384, :],
                                preferred_element_type=jnp.float32))

                cat = cat_s[i]
                sub = lax.broadcasted_iota(jnp.int32, (128, B), 0)
                for t in range(4):
                    row = cat[t:t + 1, :]
                    ck = jnp.zeros((B, 64), jnp.float32)
                    for cc in range(8):
                        oh = (row == sub + cc * 128).astype(jnp.float32)
                        ck = ck + _dotT(
                            oh, ctab_s[t, cc * 128:(cc + 1) * 128, :])
                    ev = ev + jnp.dot(ck, pw_s[64 * t:64 * (t + 1), :],
                                      preferred_element_type=jnp.float32)

                ev = _gelu(ev)
                m = jnp.mean(ev, axis=1, keepdims=True)
                va = jnp.mean((ev - m) ** 2, axis=1, keepdims=True)
                ev = ((ev - m) / jnp.sqrt(va + 1e-5) * lng_ref[...]
                      + lnb_ref[...])

                is_row = (ti == 1).astype(jnp.float32)
                is_col = _dotT(is_row, jnp.ones((1, 1), jnp.float32))
                out_ref[i] = out_ref[i] + is_col * ev


def _t_spec(shape):
    nd = len(shape)
    return pl.BlockSpec((TB,) + shape[1:],
                        lambda g: (g,) + (0,) * (nd - 1))


def _w_spec(shape):
    nd = len(shape)
    return pl.BlockSpec(shape, lambda g, _n=nd: (0,) * _n)


_ANY = pl.BlockSpec(memory_space=pltpu.MemorySpace.HBM)


def kernel(token_ids, cat_feats, num_feats, time_feats, tok_table,
           cat_tables, num_norm_g, num_norm_b, time_norm_g, time_norm_b,
           num_w1, num_b1, num_w2, num_b2, time_w1, time_b1, time_w2,
           time_b2, proj_w, proj_b, proj_ln_g, proj_ln_b, type_table,
           case_table, event_scale, type_scale, case_scale):
    ids_flat = jnp.transpose(token_ids, (1, 0)).reshape(BT).astype(jnp.int32)
    base = _sc_gather()(ids_flat, tok_table)

    ttab = jnp.pad(type_table * type_scale, ((0, 1), (0, 0)))
    cstab = case_table * case_scale
    lng = proj_ln_g * event_scale
    lnb = proj_ln_b * event_scale
    ctab = jnp.pad(cat_tables, ((0, 0), (0, 24), (0, 0)))

    r2 = lambda a: a.reshape(1, -1)
    c2 = lambda a: a.reshape(-1, 1)
    out = pl.pallas_call(
        _tc_body,
        grid=(GRID,),
        in_specs=[
            _t_spec((T, 1, B)), _t_spec((T, B, D)),
            _w_spec((8, D)), _w_spec((32, D)),
            _ANY, _ANY, _ANY, _ANY, _ANY,
            _w_spec((16, 64)), _w_spec((1, 64)),
            _w_spec((64, 64)), _w_spec((1, 64)),
            _w_spec((16, 1)), _w_spec((16, 1)),
            _w_spec((8, 64)), _w_spec((1, 64)),
            _w_spec((64, 64)), _w_spec((1, 64)),
            _w_spec((8, 1)), _w_spec((8, 1)),
            _w_spec((1, D)), _w_spec((1, D)), _w_spec((1, D)),
        ],
        out_specs=_t_spec((T, B, D)),
        out_shape=jax.ShapeDtypeStruct((T, B, D), jnp.float32),
        scratch_shapes=[
            pltpu.VMEM((1, B), jnp.float32),
            pltpu.VMEM((4, 1024, 64), jnp.float32),
            pltpu.VMEM((384, D), jnp.float32),
            pltpu.VMEM((TB, 4, B), jnp.int32),
            pltpu.VMEM((TB, 16, B), jnp.float32),
            pltpu.VMEM((TB, 8, B), jnp.float32),
            pltpu.SemaphoreType.DMA,
            pltpu.SemaphoreType.DMA,
            pltpu.SemaphoreType.DMA,
        ],
        compiler_params=pltpu.CompilerParams(
            dimension_semantics=("arbitrary",)),
    )(
        jnp.transpose(token_ids, (1, 0)).reshape(T, 1, B).astype(jnp.int32),
        base.reshape(T, B, D),
        ttab, cstab,
        jnp.transpose(cat_feats, (1, 2, 0)).astype(jnp.int32),
        jnp.transpose(num_feats, (1, 2, 0)),
        jnp.transpose(time_feats, (1, 2, 0)),
        ctab, proj_w,
        num_w1, r2(num_b1), num_w2, r2(num_b2),
        c2(num_norm_g), c2(num_norm_b),
        time_w1, r2(time_b1), time_w2, r2(time_b2),
        c2(time_norm_g), c2(time_norm_b),
        r2(proj_b), r2(lng), r2(lnb),
    )
    return jnp.transpose(out, (1, 0, 2))
